# baseline (speedup 1.0000x reference)
import jax
import jax.numpy as jnp
from jax.experimental import pallas as pl
from jax.experimental.pallas import tpu as pltpu


def _scale8_t_multi(*refs):
    o_ref = refs[-1]
    in_refs = refs[:-1]
    b = pl.program_id(0)
    for i, r in enumerate(in_refs):
        @pl.when(b == i)
        def _(r=r):
            o_ref[...] = jnp.swapaxes(r[0], 0, 1)[None] * 8.0


def kernel(x):
    B, C, H, W = x.shape
    L = H * W
    xt = jnp.transpose(x, (0, 2, 3, 1)).reshape(B, L, C)
    parts = [jax.lax.slice_in_dim(xt, i, i + 1, axis=0) for i in range(B)]
    out = pl.pallas_call(
        _scale8_t_multi,
        grid=(B,),
        in_specs=[pl.BlockSpec(memory_space=pltpu.MemorySpace.VMEM)] * B,
        out_specs=pl.BlockSpec((1, C, L), lambda b: (b, 0, 0)),
        out_shape=jax.ShapeDtypeStruct((B, C, L), x.dtype),
    )(*parts)
    return out


# speedup vs baseline: 2.0835x; 2.0835x over previous
import jax
import jax.numpy as jnp
from jax.experimental import pallas as pl


def _scale8_t_block(x_ref, o_ref):
    o_ref[...] = jnp.swapaxes(x_ref[...], 1, 2) * 8.0


def kernel(x):
    B, C, H, W = x.shape
    L = H * W
    xt = jnp.transpose(x, (0, 2, 3, 1)).reshape(B, L, C)
    b_blk = 4
    l_blk = L // 2
    out = pl.pallas_call(
        _scale8_t_block,
        grid=(B // b_blk, 2),
        in_specs=[pl.BlockSpec((b_blk, l_blk, C), lambda b, j: (b, j, 0))],
        out_specs=pl.BlockSpec((b_blk, C, l_blk), lambda b, j: (b, 0, j)),
        out_shape=jax.ShapeDtypeStruct((B, C, L), x.dtype),
    )(xt)
    return out


# speedup vs baseline: 2.4292x; 1.1659x over previous
import jax
import jax.numpy as jnp
from jax.experimental import pallas as pl


def _scale8_t_block(x_ref, o_ref):
    o_ref[...] = jnp.swapaxes(x_ref[...], 1, 2) * 8.0


def kernel(x):
    B, C, H, W = x.shape
    L = H * W
    xt = jnp.transpose(x, (0, 2, 3, 1)).reshape(B, L, C)
    b_blk = 4
    out = pl.pallas_call(
        _scale8_t_block,
        grid=(B // b_blk,),
        in_specs=[pl.BlockSpec((b_blk, L, C), lambda b: (b, 0, 0))],
        out_specs=pl.BlockSpec((b_blk, C, L), lambda b: (b, 0, 0)),
        out_shape=jax.ShapeDtypeStruct((B, C, L), x.dtype),
    )(xt)
    return out
